# baseline (speedup 1.0000x reference)
import functools

import jax
import jax.numpy as jnp
from jax import lax
from jax.experimental import pallas as pl
from jax.experimental.pallas import tpu as pltpu
from jax.experimental.pallas import tpu_sc as plsc

T = 16384
D = 2048
L = 16
NC = 2
NS = 16
NW = NC * NS
RW = T // NW
C = 16
NCHUNK = RW // C


def _sc_kernel(x_hbm, idx_hbm, emb_hbm, out_hbm, idx_v, row_v, buf_v, sem):
    wid = lax.axis_index("s") * NC + lax.axis_index("c")
    base = wid * RW
    pltpu.sync_copy(idx_hbm, idx_v)
    pltpu.async_copy(emb_hbm.at[idx_v], row_v, sem).wait()

    def chunk(g, carry):
        row0 = base + g * C
        pltpu.sync_copy(x_hbm.at[pl.ds(row0, C)], buf_v)

        def col(k, c2):
            off = k * L
            rvec = row_v[0, pl.ds(off, L)]
            for r in range(C):
                buf_v[r, pl.ds(off, L)] = buf_v[r, pl.ds(off, L)] + rvec
            return c2

        lax.fori_loop(0, D // L, col, 0)
        pltpu.sync_copy(buf_v, out_hbm.at[pl.ds(row0, C)])
        return carry

    lax.fori_loop(0, NCHUNK, chunk, 0)


@functools.partial(jax.jit, static_argnames=())
def kernel(input_features, modality_indices, embedding_weight):
    mesh = plsc.VectorSubcoreMesh(core_axis_name="c", subcore_axis_name="s")
    out = pl.kernel(
        _sc_kernel,
        mesh=mesh,
        out_type=jax.ShapeDtypeStruct((T, D), jnp.float32),
        scratch_types=[
            pltpu.VMEM((1,), jnp.int32),
            pltpu.VMEM((1, D), jnp.float32),
            pltpu.VMEM((C, D), jnp.float32),
            pltpu.SemaphoreType.DMA,
        ],
    )(input_features, modality_indices, embedding_weight)
    return out.reshape(1, T, D)


# speedup vs baseline: 1.5796x; 1.5796x over previous
import functools

import jax
import jax.numpy as jnp
from jax import lax
from jax.experimental import pallas as pl
from jax.experimental.pallas import tpu as pltpu
from jax.experimental.pallas import tpu_sc as plsc

T = 16384
D = 2048
L = 16
NC = 2
NS = 16
NW = NC * NS
RW = T // NW
C = 8
NCHUNK = RW // C


def _sc_kernel(x_hbm, idx_hbm, emb_hbm, out_hbm,
               idx_v, row_v, in0, in1, ou0, ou1, si0, si1, so0, so1, sg):
    wid = lax.axis_index("s") * NC + lax.axis_index("c")
    base = wid * RW
    pltpu.sync_copy(idx_hbm, idx_v)
    pltpu.async_copy(emb_hbm.at[idx_v], row_v, sg).wait()
    pltpu.async_copy(x_hbm.at[pl.ds(base, C)], in0, si0)
    pltpu.async_copy(x_hbm.at[pl.ds(base + C, C)], in1, si1)

    def step(g, ib, ob, si, so):
        pltpu.make_async_copy(x_hbm.at[pl.ds(0, C)], ib, si).wait()

        @pl.when(g >= 2)
        def _():
            pltpu.make_async_copy(ob, out_hbm.at[pl.ds(0, C)], so).wait()

        def col(k, c2):
            off = k * L
            rvec = row_v[0, pl.ds(off, L)]
            for r in range(C):
                ob[r, pl.ds(off, L)] = ib[r, pl.ds(off, L)] + rvec
            return c2

        lax.fori_loop(0, D // L, col, 0)
        row0 = base + g * C
        pltpu.async_copy(ob, out_hbm.at[pl.ds(row0, C)], so)

        @pl.when(g + 2 < NCHUNK)
        def _():
            pltpu.async_copy(x_hbm.at[pl.ds(row0 + 2 * C, C)], ib, si)

    def outer(p, carry):
        step(2 * p, in0, ou0, si0, so0)
        step(2 * p + 1, in1, ou1, si1, so1)
        return carry

    lax.fori_loop(0, NCHUNK // 2, outer, 0)
    pltpu.make_async_copy(ou0, out_hbm.at[pl.ds(0, C)], so0).wait()
    pltpu.make_async_copy(ou1, out_hbm.at[pl.ds(0, C)], so1).wait()


def kernel(input_features, modality_indices, embedding_weight):
    mesh = plsc.VectorSubcoreMesh(core_axis_name="c", subcore_axis_name="s")
    out = pl.kernel(
        _sc_kernel,
        mesh=mesh,
        out_type=jax.ShapeDtypeStruct((T, D), jnp.float32),
        scratch_types=[
            pltpu.VMEM((1,), jnp.int32),
            pltpu.VMEM((1, D), jnp.float32),
            pltpu.VMEM((C, D), jnp.float32),
            pltpu.VMEM((C, D), jnp.float32),
            pltpu.VMEM((C, D), jnp.float32),
            pltpu.VMEM((C, D), jnp.float32),
            pltpu.SemaphoreType.DMA,
            pltpu.SemaphoreType.DMA,
            pltpu.SemaphoreType.DMA,
            pltpu.SemaphoreType.DMA,
            pltpu.SemaphoreType.DMA,
        ],
    )(input_features, modality_indices, embedding_weight)
    return out.reshape(1, T, D)


# speedup vs baseline: 2.1285x; 1.3475x over previous
import jax
import jax.numpy as jnp
from jax import lax
from jax.experimental import pallas as pl
from jax.experimental.pallas import tpu as pltpu
from jax.experimental.pallas import tpu_sc as plsc

T = 16384
D = 2048
BT = 1024


def _sc_lookup(idx_hbm, emb_hbm, row_hbm, idx_v, row_v, sem):
    wid = lax.axis_index("s") * 2 + lax.axis_index("c")

    @pl.when(wid == 0)
    def _():
        pltpu.sync_copy(idx_hbm, idx_v)
        pltpu.async_copy(emb_hbm.at[idx_v], row_v, sem).wait()
        pltpu.sync_copy(row_v, row_hbm)


def _tc_add(row_ref, x_ref, o_ref):
    o_ref[0] = x_ref[...] + row_ref[...]


def kernel(input_features, modality_indices, embedding_weight):
    mesh = plsc.VectorSubcoreMesh(core_axis_name="c", subcore_axis_name="s")
    row = pl.kernel(
        _sc_lookup,
        mesh=mesh,
        out_type=jax.ShapeDtypeStruct((1, D), jnp.float32),
        scratch_types=[
            pltpu.VMEM((1,), jnp.int32),
            pltpu.VMEM((1, D), jnp.float32),
            pltpu.SemaphoreType.DMA,
        ],
    )(modality_indices, embedding_weight)

    out = pl.pallas_call(
        _tc_add,
        grid=(T // BT,),
        in_specs=[
            pl.BlockSpec((1, D), lambda i: (0, 0)),
            pl.BlockSpec((BT, D), lambda i: (i, 0)),
        ],
        out_specs=pl.BlockSpec((1, BT, D), lambda i: (0, i, 0)),
        out_shape=jax.ShapeDtypeStruct((1, T, D), input_features.dtype),
        compiler_params=pltpu.CompilerParams(
            dimension_semantics=("arbitrary",),
        ),
    )(row, input_features)
    return out


# speedup vs baseline: 2.6329x; 1.2370x over previous
import jax
import jax.numpy as jnp
from jax.experimental import pallas as pl
from jax.experimental.pallas import tpu as pltpu

T = 16384
D = 2048
BT = 1024


def _add_kernel(idx_ref, emb_ref, x_ref, o_ref):
    i = idx_ref[0]
    row = emb_ref[pl.ds(i, 1), :]
    o_ref[0] = x_ref[...] + row


def kernel(input_features, modality_indices, embedding_weight):
    grid = (T // BT,)
    out = pl.pallas_call(
        _add_kernel,
        grid_spec=pltpu.PrefetchScalarGridSpec(
            num_scalar_prefetch=1,
            grid=grid,
            in_specs=[
                pl.BlockSpec((4, D), lambda i, idx: (0, 0)),
                pl.BlockSpec((BT, D), lambda i, idx: (i, 0)),
            ],
            out_specs=pl.BlockSpec((1, BT, D), lambda i, idx: (0, i, 0)),
        ),
        out_shape=jax.ShapeDtypeStruct((1, T, D), input_features.dtype),
        compiler_params=pltpu.CompilerParams(
            dimension_semantics=("arbitrary",),
        ),
    )(modality_indices, embedding_weight, input_features)
    return out
